# Initial kernel scaffold; baseline (speedup 1.0000x reference)
import jax
import jax.numpy as jnp
from jax.experimental import pallas as pl


def kernel(x, alpha, dyt_w, dyt_b, W_in, b_in, W_out, b_out, A, Bp, Cp, conv_w):
    raise NotImplementedError("write your pallas kernel here")


# speedup vs baseline: 1.8508x; 1.8508x over previous
import functools

import jax
import jax.numpy as jnp
from jax import lax
from jax.experimental import pallas as pl
from jax.experimental.pallas import tpu as pltpu
import jax.experimental.pallas.tpu_sc as plsc

L = 4096
DIM = 768
C = 1536
N = 16
K = 1228
KP = 1280
T = 128
B = 2
DUMMY = 2 * L
NR = B * KP
RIDX_N = NR + 16


def _k1_center(al_ref, w_ref, b_ref, xc_ref, win_ref, bin_ref, cn_ref):
    xn = jnp.tanh(al_ref[0, 0] * xc_ref[...]) * w_ref[...] + b_ref[...]
    xp = jax.lax.dot_general(xn, win_ref[...], (((1,), (1,)), ((), ())),
                             preferred_element_type=jnp.float32) + bin_ref[...]
    nrm = jnp.sqrt(jnp.sum(xp * xp, axis=-1, keepdims=True))
    cn_ref[...] = xp / jnp.clip(nrm, 1e-12, None)


def _k2_proj(al_ref, w_ref, b_ref, x_ref, win_ref, bin_ref, cn_ref,
             xp_ref, sim_ref):
    x = x_ref[...].reshape(256, DIM)
    xn = jnp.tanh(al_ref[0, 0] * x) * w_ref[...] + b_ref[...]
    xp = jax.lax.dot_general(xn, win_ref[...], (((1,), (1,)), ((), ())),
                             preferred_element_type=jnp.float32) + bin_ref[...]
    xp_ref[...] = xp.reshape(1, 256, C)
    x2 = (xp * xp).reshape(256, 12, 128)
    nps = jnp.sum(x2, axis=-1)
    nr = nps[:, 0]
    for j in range(1, 12):
        nr = nr + nps[:, j]
    nrm = jnp.sqrt(nr).reshape(256, 1)
    xu = xp / jnp.clip(nrm, 1e-12, None)
    xub = xu.astype(jnp.bfloat16).astype(jnp.float32)
    cnb = cn_ref[...].reshape(1, C).astype(jnp.bfloat16).astype(jnp.float32)
    p3 = (xub * cnb).reshape(256, 12, 128)
    ps = jnp.sum(p3, axis=-1)
    s = ps[:, 0]
    for j in range(1, 12):
        s = s + ps[:, j]
    sim_ref[...] = s.reshape(1, 1, 256)


def _k3_rank(si_ref, sj_ref, rank_ref):
    i = pl.program_id(1)
    j = pl.program_id(2)

    @pl.when(j == 0)
    def _():
        rank_ref[...] = jnp.zeros_like(rank_ref)

    si = si_ref[...].reshape(256, 1)
    sj = sj_ref[...].reshape(1, 1024)
    ig = i * 256 + jax.lax.broadcasted_iota(jnp.int32, (256, 1), 0)
    jg = j * 1024 + jax.lax.broadcasted_iota(jnp.int32, (1, 1024), 1)
    beat = (sj > si) | ((sj == si) & (jg < ig))
    cnt = jnp.sum(beat.astype(jnp.int32), axis=1)
    rank_ref[...] += cnt.reshape(1, 1, 256)


def _sc_invert(rank_hbm, ridx_hbm, fillv, rv, dbi, dbv, sem):
    c = lax.axis_index("c")
    s = lax.axis_index("s")
    for q in range(5):
        fillv[pl.ds(q * 16, 16)] = jnp.full((16,), DUMMY, jnp.int32)
    pltpu.sync_copy(fillv, ridx_hbm.at[pl.ds(c * KP + s * 80, 80)])
    plsc.subcore_barrier()
    for j in range(4):
        g0 = c * L + s * 256 + j * 64
        pltpu.sync_copy(rank_hbm.at[pl.ds(g0, 64)], rv)
        for q in range(4):
            r16 = rv[pl.ds(q * 16, 16)]
            val16 = g0 + q * 16 + lax.iota(jnp.int32, 16)
            dst16 = jnp.where(r16 < K, c * KP + r16, NR)
            dbi[pl.ds(q * 16, 16)] = dst16
            dbv[pl.ds(q * 16, 16)] = val16
        pltpu.async_copy(dbv, ridx_hbm.at[dbi], sem).wait()


def _sc_gather(tab_hbm, ridx_hbm, out_hbm, idxv, rows, sem):
    c = lax.axis_index("c")
    s = lax.axis_index("s")
    w = c * 16 + s
    pltpu.sync_copy(ridx_hbm.at[pl.ds(w * 80, 80)], idxv)
    for q in range(5):
        idxv[pl.ds(q * 16, 16)] = jnp.minimum(
            idxv[pl.ds(q * 16, 16)], DUMMY - 1)
    pltpu.async_copy(tab_hbm.at[idxv], rows, sem).wait()
    pltpu.sync_copy(rows, out_hbm.at[pl.ds(w * 80, 80)])


def _sc_scatter(x_hbm, ridx_hbm, yf_hbm, out_hbm, buf, idxv, rows, sem):
    c = lax.axis_index("c")
    s = lax.axis_index("s")
    for j in range(4):
        r0 = c * L + s * 256 + j * 64
        pltpu.sync_copy(x_hbm.at[pl.ds(r0, 64)], buf)
        pltpu.sync_copy(buf, out_hbm.at[pl.ds(r0, 64)])
    plsc.subcore_barrier()
    w0 = c * KP + s * 80
    pltpu.sync_copy(ridx_hbm.at[pl.ds(w0, 80)], idxv)
    pltpu.sync_copy(yf_hbm.at[pl.ds(w0, 80)], rows)
    pltpu.async_copy(rows, out_hbm.at[idxv], sem).wait()


def _k6a_scan(xs_ref, w4_ref, kstk_ref, apv_ref, pt_ref, m128_ref, sct_ref,
              ys_ref, ht_ref, tail_ref):
    n = pl.program_id(1)

    @pl.when(n == 0)
    def _():
        ht_ref[...] = jnp.zeros_like(ht_ref)
        tail_ref[...] = jnp.zeros_like(tail_ref)

    xblk = xs_ref[...].reshape(T, C)
    ext = jnp.concatenate([tail_ref[...], xblk], axis=0)
    w4 = w4_ref[...]
    xc = (w4[0:1, :] * ext[5:5 + T] + w4[1:2, :] * ext[6:6 + T]
          + w4[2:3, :] * ext[7:7 + T] + w4[3:4, :] * ext[8:8 + T])
    ht = ht_ref[...]
    sct = sct_ref[...]
    g2 = (sct[:, None, :] * ht[None, :, :]).reshape(N * N, C)
    y = jax.lax.dot_general(apv_ref[...], g2, (((1,), (0,)), ((), ())),
                            preferred_element_type=jnp.float32)
    for m in range(N):
        km = kstk_ref[m]
        cm = jax.lax.dot_general(km, xc, (((1,), (0,)), ((), ())),
                                 preferred_element_type=jnp.float32)
        y = y + cm * sct[m:m + 1, :]
    ys_ref[...] = y.reshape(1, T, C)
    ht_ref[...] = (jnp.dot(m128_ref[...], ht,
                           preferred_element_type=jnp.float32)
                   + jax.lax.dot_general(pt_ref[...], xc,
                                         (((1,), (0,)), ((), ())),
                                         preferred_element_type=jnp.float32))
    tail_ref[...] = ext[T:T + 8]


def _k6b_proj(ys_ref, wout_ref, bout_ref, xr_ref, yf_ref):
    y = ys_ref[...].reshape(T, C)
    o = jax.lax.dot_general(y, wout_ref[...], (((1,), (1,)), ((), ())),
                            preferred_element_type=jnp.float32)
    yf_ref[...] = (o + bout_ref[...] + xr_ref[...].reshape(T, DIM)
                   ).reshape(1, T, DIM)


def _k7_base(x_ref, o_ref):
    r = pl.program_id(0)
    o_ref[...] = jnp.where(r >= 1024, jnp.zeros_like(x_ref), x_ref[...])


def _k7_scatter(idx_ref, yf_ref, base_ref, o_ref):
    del idx_ref, base_ref
    o_ref[...] = yf_ref[...]


def _f32(x):
    return jnp.asarray(x, jnp.float32)


def kernel(x, alpha, dyt_w, dyt_b, W_in, b_in, W_out, b_out, A, Bp, Cp,
           conv_w):
    sigB = jax.nn.sigmoid(Bp).reshape(N)
    sigC = jax.nn.sigmoid(Cp)
    sct = sigC.T
    S = jnp.eye(N, dtype=jnp.float32)[None]
    Am = A
    while S.shape[0] < T:
        S = jnp.concatenate([S, jnp.einsum('dmn,nk->dmk', S, Am)], axis=0)
        Am = jnp.dot(Am, Am)
    S = S[:T]
    a_t1 = jnp.concatenate(
        [S[1:], jnp.dot(S[-1], A)[None]], axis=0)
    apv = a_t1.reshape(T, N * N)
    m128 = a_t1[-1]
    kv = jnp.einsum('dmn,n->dm', S, sigB)
    pt = kv[::-1].T
    d = (jnp.arange(T)[:, None] - jnp.arange(T)[None, :])
    kstk = jnp.where((d >= 0)[None], kv[jnp.clip(d, 0, T - 1)].transpose(
        2, 0, 1), 0.0).astype(jnp.float32)
    w4 = conv_w[:, 0, :].T

    al = alpha.reshape(1, 1)
    dw = dyt_w.reshape(1, DIM)
    db = dyt_b.reshape(1, DIM)
    bi = b_in.reshape(1, C)
    bo = b_out.reshape(1, DIM)

    xc = x[:, L // 2, :]
    cn = pl.pallas_call(
        _k1_center,
        out_shape=jax.ShapeDtypeStruct((B, C), jnp.float32),
    )(al, dw, db, xc, W_in, bi)

    xp, sim = pl.pallas_call(
        _k2_proj,
        grid=(B, L // 256),
        in_specs=[
            pl.BlockSpec((1, 1), lambda b, l: (0, 0)),
            pl.BlockSpec((1, DIM), lambda b, l: (0, 0)),
            pl.BlockSpec((1, DIM), lambda b, l: (0, 0)),
            pl.BlockSpec((1, 256, DIM), lambda b, l: (b, l, 0)),
            pl.BlockSpec((C, DIM), lambda b, l: (0, 0)),
            pl.BlockSpec((1, C), lambda b, l: (0, 0)),
            pl.BlockSpec((1, 1, C), lambda b, l: (b, 0, 0)),
        ],
        out_specs=[
            pl.BlockSpec((1, 256, C), lambda b, l: (b, l, 0)),
            pl.BlockSpec((1, 1, 256), lambda b, l: (b, 0, l)),
        ],
        out_shape=[
            jax.ShapeDtypeStruct((B, L, C), jnp.float32),
            jax.ShapeDtypeStruct((B, 1, L), jnp.float32),
        ],
    )(al, dw, db, x, W_in, bi, cn.reshape(B, 1, C))

    rank = pl.pallas_call(
        _k3_rank,
        grid=(B, L // 256, L // 1024),
        in_specs=[
            pl.BlockSpec((1, 1, 256), lambda b, i, j: (b, 0, i)),
            pl.BlockSpec((1, 1, 1024), lambda b, i, j: (b, 0, j)),
        ],
        out_specs=pl.BlockSpec((1, 1, 256), lambda b, i, j: (b, 0, i)),
        out_shape=jax.ShapeDtypeStruct((B, 1, L), jnp.int32),
    )(sim, sim)

    mesh = plsc.VectorSubcoreMesh(core_axis_name="c", subcore_axis_name="s")
    ridx = pl.kernel(
        _sc_invert, mesh=mesh,
        out_type=jax.ShapeDtypeStruct((RIDX_N,), jnp.int32),
        scratch_types=[
            pltpu.VMEM((80,), jnp.int32),
            pltpu.VMEM((64,), jnp.int32),
            pltpu.VMEM((64,), jnp.int32),
            pltpu.VMEM((64,), jnp.int32),
            pltpu.SemaphoreType.DMA,
        ],
    )(rank.reshape(B * L))

    xs = pl.kernel(
        _sc_gather, mesh=mesh,
        out_type=jax.ShapeDtypeStruct((NR, C), jnp.float32),
        scratch_types=[
            pltpu.VMEM((80,), jnp.int32),
            pltpu.VMEM((80, C), jnp.float32),
            pltpu.SemaphoreType.DMA,
        ],
    )(xp.reshape(B * L, C), ridx)
    xr = pl.kernel(
        _sc_gather, mesh=mesh,
        out_type=jax.ShapeDtypeStruct((NR, DIM), jnp.float32),
        scratch_types=[
            pltpu.VMEM((80,), jnp.int32),
            pltpu.VMEM((80, DIM), jnp.float32),
            pltpu.SemaphoreType.DMA,
        ],
    )(x.reshape(B * L, DIM), ridx)

    xs3 = xs.reshape(B, KP, C)
    ys = pl.pallas_call(
        _k6a_scan,
        grid=(B, KP // T),
        in_specs=[
            pl.BlockSpec((1, T, C), lambda b, n: (b, n, 0)),
            pl.BlockSpec((4, C), lambda b, n: (0, 0)),
            pl.BlockSpec((N, T, T), lambda b, n: (0, 0, 0)),
            pl.BlockSpec((T, N * N), lambda b, n: (0, 0)),
            pl.BlockSpec((N, T), lambda b, n: (0, 0)),
            pl.BlockSpec((N, N), lambda b, n: (0, 0)),
            pl.BlockSpec((N, C), lambda b, n: (0, 0)),
        ],
        out_specs=pl.BlockSpec((1, T, C), lambda b, n: (b, n, 0)),
        out_shape=jax.ShapeDtypeStruct((B, KP, C), jnp.float32),
        scratch_shapes=[
            pltpu.VMEM((N, C), jnp.float32),
            pltpu.VMEM((8, C), jnp.float32),
        ],
    )(xs3, w4, kstk, apv, pt, m128, sct)

    xr3 = xr.reshape(B, KP, DIM)
    yf = pl.pallas_call(
        _k6b_proj,
        grid=(B, KP // T),
        in_specs=[
            pl.BlockSpec((1, T, C), lambda b, n: (b, n, 0)),
            pl.BlockSpec((DIM, C), lambda b, n: (0, 0)),
            pl.BlockSpec((1, DIM), lambda b, n: (0, 0)),
            pl.BlockSpec((1, T, DIM), lambda b, n: (b, n, 0)),
        ],
        out_specs=pl.BlockSpec((1, T, DIM), lambda b, n: (b, n, 0)),
        out_shape=jax.ShapeDtypeStruct((B, KP, DIM), jnp.float32),
    )(ys, W_out, bo, xr3)
    yf = yf.reshape(NR, DIM)

    out = pl.kernel(
        _sc_scatter, mesh=mesh,
        out_type=jax.ShapeDtypeStruct((DUMMY + 8, DIM), jnp.float32),
        scratch_types=[
            pltpu.VMEM((64, DIM), jnp.float32),
            pltpu.VMEM((80,), jnp.int32),
            pltpu.VMEM((80, DIM), jnp.float32),
            pltpu.SemaphoreType.DMA,
        ],
    )(x.reshape(B * L, DIM), ridx, yf)
    return out[:B * L].reshape(B, L, DIM)
